# Initial kernel scaffold; baseline (speedup 1.0000x reference)
import jax
import jax.numpy as jnp
from jax.experimental import pallas as pl


def kernel(SV, D, idx):
    raise NotImplementedError("write your pallas kernel here")


# speedup vs baseline: 7.5552x; 7.5552x over previous
import functools

import jax
import jax.numpy as jnp
from jax import lax
from jax.experimental import pallas as pl
from jax.experimental.pallas import tpu as pltpu
from jax.experimental.pallas import tpu_sc as plsc


def _matmul_kernel(svt_ref, d_ref, e_ref):
    e_ref[...] = lax.dot_general(
        svt_ref[...], d_ref[...],
        dimension_numbers=(((0,), (0,)), ((), ())),
        preferred_element_type=jnp.float32)


def _decode_table(SVt, D):
    rank, vocab = SVt.shape
    embed = D.shape[1]
    blk = 4096
    return pl.pallas_call(
        _matmul_kernel,
        grid=(pl.cdiv(vocab, blk),),
        in_specs=[
            pl.BlockSpec((rank, blk), lambda i: (0, i)),
            pl.BlockSpec((rank, embed), lambda i: (0, 0)),
        ],
        out_specs=pl.BlockSpec((blk, embed), lambda i: (i, 0)),
        out_shape=jax.ShapeDtypeStruct((vocab, embed), jnp.float32),
    )(SVt, D)


def _make_gather(n, embed, n_workers, chunk, nbuf):
    per_w = n // n_workers
    n_chunks = per_w // chunk
    n_groups = n_chunks // nbuf
    assert per_w % chunk == 0 and n_chunks % nbuf == 0 and n_groups >= 2
    mesh = plsc.VectorSubcoreMesh(core_axis_name="c", subcore_axis_name="s")
    info = plsc.get_sparse_core_info()
    nc = info.num_cores

    scratch = ([pltpu.VMEM((per_w,), jnp.int32)]
               + [pltpu.VMEM((chunk, embed), jnp.float32)] * nbuf
               + [pltpu.SemaphoreType.DMA] * (2 * nbuf))

    @functools.partial(
        pl.kernel,
        mesh=mesh,
        out_type=jax.ShapeDtypeStruct((n, embed), jnp.float32),
        scratch_types=scratch,
    )
    def gather(e_hbm, idx_hbm, out_hbm, idx_v, *bufsems):
        bufs = bufsems[:nbuf]
        gsems = bufsems[nbuf:2 * nbuf]
        wsems = bufsems[2 * nbuf:]
        wid = lax.axis_index("s") * nc + lax.axis_index("c")
        base = wid * per_w

        def fire_gather(c, b):
            pltpu.async_copy(
                e_hbm.at[idx_v.at[pl.ds(c * chunk, chunk)]], bufs[b], gsems[b])

        def wait_gather(c, b):
            pltpu.make_async_copy(
                e_hbm.at[idx_v.at[pl.ds(c * chunk, chunk)]], bufs[b],
                gsems[b]).wait()

        def fire_write(c, b):
            pltpu.async_copy(
                bufs[b], out_hbm.at[pl.ds(base + c * chunk, chunk), :],
                wsems[b])

        def wait_write(c, b):
            pltpu.make_async_copy(
                bufs[b], out_hbm.at[pl.ds(base + c * chunk, chunk), :],
                wsems[b]).wait()

        pltpu.sync_copy(idx_hbm.at[pl.ds(base, per_w)], idx_v)

        for b in range(nbuf - 1):
            fire_gather(b, b)

        wait_gather(0, 0)
        fire_write(0, 0)
        fire_gather(nbuf - 1, nbuf - 1)
        for b in range(1, nbuf):
            wait_gather(b, b)
            fire_write(b, b)
            wait_write(b - 1, (b - 1) % nbuf)
            fire_gather(b + nbuf - 1, (b - 1) % nbuf)

        def body(p, carry):
            c0 = p * nbuf
            for b in range(nbuf):
                c = c0 + b
                bp = (b + nbuf - 1) % nbuf
                wait_gather(c, b)
                fire_write(c, b)
                wait_write(c - 1, bp)
                fire_gather(c + nbuf - 1, bp)
            return carry

        lax.fori_loop(1, n_groups - 1, body, 0)

        c0 = (n_groups - 1) * nbuf
        wait_gather(c0, 0)
        fire_write(c0, 0)
        wait_write(c0 - 1, nbuf - 1)
        fire_gather(c0 + nbuf - 1, nbuf - 1)
        for b in range(1, nbuf):
            wait_gather(c0 + b, b)
            fire_write(c0 + b, b)
        for b in range(nbuf):
            wait_write(c0 + b, b)

    return gather


def kernel(SV, D, idx):
    b, l = idx.shape
    embed = D.shape[1]
    n = b * l
    E = _decode_table(SV.T, D)
    idx_flat = idx.T.reshape(-1).astype(jnp.int32)
    gather = _make_gather(n, embed, n_workers=32, chunk=128, nbuf=5)
    out = gather(E, idx_flat)
    return out.reshape(l, b, embed).transpose(1, 0, 2)


# speedup vs baseline: 7.9562x; 1.0531x over previous
import functools

import jax
import jax.numpy as jnp
from jax import lax
from jax.experimental import pallas as pl
from jax.experimental.pallas import tpu as pltpu
from jax.experimental.pallas import tpu_sc as plsc


def _matmul_kernel(svt_ref, d_ref, e_ref):
    e_ref[...] = lax.dot_general(
        svt_ref[...], d_ref[...],
        dimension_numbers=(((0,), (0,)), ((), ())),
        preferred_element_type=jnp.float32)


def _decode_table(SVt, D):
    rank, vocab = SVt.shape
    embed = D.shape[1]
    blk = 8192
    return pl.pallas_call(
        _matmul_kernel,
        grid=(pl.cdiv(vocab, blk),),
        in_specs=[
            pl.BlockSpec((rank, blk), lambda i: (0, i)),
            pl.BlockSpec((rank, embed), lambda i: (0, 0)),
        ],
        out_specs=pl.BlockSpec((blk, embed), lambda i: (i, 0)),
        out_shape=jax.ShapeDtypeStruct((vocab, embed), jnp.float32),
    )(SVt, D)


def _make_gather(n, embed, n_workers, chunk, nbuf):
    per_w = n // n_workers
    n_chunks = per_w // chunk
    n_groups = n_chunks // nbuf
    assert per_w % chunk == 0 and n_chunks % nbuf == 0 and n_groups >= 2
    mesh = plsc.VectorSubcoreMesh(core_axis_name="c", subcore_axis_name="s")
    info = plsc.get_sparse_core_info()
    nc = info.num_cores

    scratch = ([pltpu.VMEM((per_w,), jnp.int32)]
               + [pltpu.VMEM((chunk, embed), jnp.float32)] * nbuf
               + [pltpu.SemaphoreType.DMA] * (2 * nbuf))

    @functools.partial(
        pl.kernel,
        mesh=mesh,
        out_type=jax.ShapeDtypeStruct((n, embed), jnp.float32),
        scratch_types=scratch,
    )
    def gather(e_hbm, idx_hbm, out_hbm, idx_v, *bufsems):
        bufs = bufsems[:nbuf]
        gsems = bufsems[nbuf:2 * nbuf]
        wsems = bufsems[2 * nbuf:]
        wid = lax.axis_index("s") * nc + lax.axis_index("c")
        base = wid * per_w

        def fire_gather(c, b):
            pltpu.async_copy(
                e_hbm.at[idx_v.at[pl.ds(c * chunk, chunk)]], bufs[b], gsems[b])

        def wait_gather(c, b):
            pltpu.make_async_copy(
                e_hbm.at[idx_v.at[pl.ds(c * chunk, chunk)]], bufs[b],
                gsems[b]).wait()

        def fire_write(c, b):
            pltpu.async_copy(
                bufs[b], out_hbm.at[pl.ds(base + c * chunk, chunk), :],
                wsems[b])

        def wait_write(c, b):
            pltpu.make_async_copy(
                bufs[b], out_hbm.at[pl.ds(base + c * chunk, chunk), :],
                wsems[b]).wait()

        pltpu.sync_copy(idx_hbm.at[pl.ds(base, per_w)], idx_v)

        for b in range(nbuf - 1):
            fire_gather(b, b)

        wait_gather(0, 0)
        fire_write(0, 0)
        fire_gather(nbuf - 1, nbuf - 1)
        for b in range(1, nbuf):
            wait_gather(b, b)
            fire_write(b, b)
            wait_write(b - 1, (b - 1) % nbuf)
            fire_gather(b + nbuf - 1, (b - 1) % nbuf)

        def body(p, carry):
            c0 = p * nbuf
            for b in range(nbuf):
                c = c0 + b
                bp = (b + nbuf - 1) % nbuf
                wait_gather(c, b)
                fire_write(c, b)
                wait_write(c - 1, bp)
                fire_gather(c + nbuf - 1, bp)
            return carry

        lax.fori_loop(1, n_groups - 1, body, 0)

        c0 = (n_groups - 1) * nbuf
        wait_gather(c0, 0)
        fire_write(c0, 0)
        wait_write(c0 - 1, nbuf - 1)
        fire_gather(c0 + nbuf - 1, nbuf - 1)
        for b in range(1, nbuf):
            wait_gather(c0 + b, b)
            fire_write(c0 + b, b)
        for b in range(nbuf):
            wait_write(c0 + b, b)

    return gather


def kernel(SV, D, idx):
    b, l = idx.shape
    embed = D.shape[1]
    n = b * l
    E = _decode_table(SV.T, D)
    idx_flat = idx.T.reshape(-1).astype(jnp.int32)
    gather = _make_gather(n, embed, n_workers=32, chunk=128, nbuf=5)
    out = gather(E, idx_flat)
    return out.reshape(l, b, embed).transpose(1, 0, 2)
